# baseline (speedup 1.0000x reference)
import functools

import jax
import jax.numpy as jnp
from jax import lax
from jax.experimental import pallas as pl
from jax.experimental.pallas import tpu as pltpu
from jax.experimental.pallas import tpu_sc as plsc

DIM = 64
NW = 32
BB = 128
LANES = 16


@functools.lru_cache(maxsize=None)
def _build(batch, hist, npairs):
    nblk = hist
    mesh = plsc.VectorSubcoreMesh(core_axis_name="c", subcore_axis_name="s")

    @functools.partial(
        pl.kernel,
        mesh=mesh,
        out_type=jax.ShapeDtypeStruct((hist, DIM, batch), jnp.float32),
        scratch_types=[
            pltpu.VMEM((nblk, BB), jnp.int32),
            pltpu.VMEM((2, BB), jnp.int32),
            pltpu.VMEM((2, BB, 2 * DIM), jnp.float32),
            pltpu.VMEM((2, DIM, BB), jnp.float32),
            pltpu.SemaphoreType.DMA,
            pltpu.SemaphoreType.DMA,
        ],
        compiler_params=pltpu.CompilerParams(
            use_tc_tiling_on_sc=True, needs_layout_passes=False),
    )
    def gather_kernel(idx_hbm, table_hbm, out_hbm,
                      idx_v, pidx_v, pair_v, outb_v, gsem, wsem):
        wid = lax.axis_index("s") * 2 + lax.axis_index("c")
        col0 = wid * BB
        pltpu.sync_copy(idx_hbm.at[:, pl.ds(col0, BB)], idx_v)

        iota = lax.iota(jnp.int32, LANES)

        def fire_gather(blk, buf):
            for jg in range(BB // LANES):
                v = idx_v[blk, pl.ds(jg * LANES, LANES)]
                pidx_v[buf, pl.ds(jg * LANES, LANES)] = (
                    lax.shift_right_logical(v, 1))
            pltpu.async_copy(table_hbm.at[pidx_v.at[buf]],
                             pair_v.at[buf], gsem)

        def drain_gather():
            pltpu.make_async_copy(
                table_hbm.at[pl.ds(0, BB)], pair_v.at[0], gsem).wait()

        def select_transpose(blk, buf):
            def jg_body(jg2, carry):
                for half in range(2):
                    jg = jg2 * 2 + half
                    v = idx_v[blk, pl.ds(jg * LANES, LANES)]
                    rowv = jg * LANES + iota
                    colbase = lax.shift_left(lax.bitwise_and(v, 1), 6)
                    for fc in range(DIM // LANES):
                        vals = [
                            plsc.load_gather(
                                pair_v.at[buf],
                                [rowv, colbase + (fc * LANES + u)])
                            for u in range(LANES)
                        ]
                        for u in range(LANES):
                            outb_v[buf, fc * LANES + u,
                                   pl.ds(jg * LANES, LANES)] = vals[u]
                return carry

            lax.fori_loop(0, BB // LANES // 2, jg_body, 0)

        def fire_write(blk, buf):
            pltpu.async_copy(outb_v.at[buf],
                             out_hbm.at[blk, :, pl.ds(col0, BB)], wsem)

        def drain_write():
            pltpu.make_async_copy(
                outb_v.at[0], out_hbm.at[0, :, pl.ds(0, BB)], wsem).wait()

        fire_gather(0, 0)
        fire_gather(1, 1)

        drain_gather()
        select_transpose(0, 0)
        fire_write(0, 0)
        fire_gather(2, 0)
        drain_gather()
        select_transpose(1, 1)
        fire_write(1, 1)
        fire_gather(3, 1)

        def body(i2, carry):
            for k in range(2):
                i = 2 * i2 + k
                drain_gather()
                drain_write()
                select_transpose(i, k)
                fire_write(i, k)
                fire_gather(i + 2, k)
            return carry

        lax.fori_loop(1, nblk // 2 - 1, body, 0)

        for i in (nblk - 2, nblk - 1):
            buf = i % 2
            drain_gather()
            drain_write()
            select_transpose(i, buf)
            fire_write(i, buf)
        drain_write()
        drain_write()

    return gather_kernel


def kernel(indices, W):
    batch, hist = indices.shape
    vocab, dim = W.shape
    idx_t = jnp.transpose(indices).astype(jnp.int32)
    w_pairs = jnp.reshape(W, (vocab * dim // (2 * DIM), 2 * DIM))
    out = _build(batch, hist, w_pairs.shape[0])(idx_t, w_pairs)
    return jnp.transpose(out, (2, 0, 1))


# speedup vs baseline: 1.1168x; 1.1168x over previous
import functools

import jax
import jax.numpy as jnp
from jax import lax
from jax.experimental import pallas as pl
from jax.experimental.pallas import tpu as pltpu
from jax.experimental.pallas import tpu_sc as plsc

DIM = 64
NW = 32
BB = 128
LANES = 16


@functools.lru_cache(maxsize=None)
def _build(batch, hist, npairs):
    nblk = hist
    mesh = plsc.VectorSubcoreMesh(core_axis_name="c", subcore_axis_name="s")

    @functools.partial(
        pl.kernel,
        mesh=mesh,
        out_type=jax.ShapeDtypeStruct((hist, DIM, batch), jnp.float32),
        scratch_types=[
            pltpu.VMEM((nblk, BB), jnp.int32),
            pltpu.VMEM((2, BB), jnp.int32),
            pltpu.VMEM((2, BB, 2 * DIM), jnp.float32),
            pltpu.VMEM((2, DIM, BB), jnp.float32),
            pltpu.SemaphoreType.DMA,
            pltpu.SemaphoreType.DMA,
        ],
        compiler_params=pltpu.CompilerParams(
            use_tc_tiling_on_sc=True, needs_layout_passes=False),
    )
    def gather_kernel(idx_hbm, table_hbm, out_hbm,
                      idx_v, pidx_v, pair_v, outb_v, gsem, wsem):
        wid = lax.axis_index("s") * 2 + lax.axis_index("c")
        col0 = wid * BB
        pltpu.sync_copy(idx_hbm.at[:, pl.ds(col0, BB)], idx_v)

        iota = lax.iota(jnp.int32, LANES)

        def fire_gather(blk, buf):
            for jg in range(BB // LANES):
                v = idx_v[blk, pl.ds(jg * LANES, LANES)]
                pidx_v[buf, pl.ds(jg * LANES, LANES)] = (
                    lax.shift_right_logical(v, 1))
            pltpu.async_copy(table_hbm.at[pidx_v.at[buf]],
                             pair_v.at[buf], gsem)

        def drain_gather():
            pltpu.make_async_copy(
                table_hbm.at[pl.ds(0, BB)], pair_v.at[0], gsem).wait()

        def select_transpose(blk, buf):
            rots = [lax.bitwise_and(iota + s, LANES - 1)
                    for s in range(LANES)]

            def jg_body(jg, carry):
                v = idx_v[blk, pl.ds(jg * LANES, LANES)]
                rowv = jg * LANES + iota
                colbase = lax.shift_left(lax.bitwise_and(v, 1), 6)
                for fc in range(DIM // LANES):
                    for s in range(LANES):
                        fvec = fc * LANES + rots[s]
                        vals = plsc.load_gather(
                            pair_v.at[buf], [rowv, colbase + fvec])
                        plsc.store_scatter(
                            outb_v.at[buf], [fvec, rowv], vals)
                return carry

            lax.fori_loop(0, BB // LANES, jg_body, 0)

        def fire_write(blk, buf):
            pltpu.async_copy(outb_v.at[buf],
                             out_hbm.at[blk, :, pl.ds(col0, BB)], wsem)

        def drain_write():
            pltpu.make_async_copy(
                outb_v.at[0], out_hbm.at[0, :, pl.ds(0, BB)], wsem).wait()

        fire_gather(0, 0)
        fire_gather(1, 1)

        drain_gather()
        select_transpose(0, 0)
        fire_write(0, 0)
        fire_gather(2, 0)
        drain_gather()
        select_transpose(1, 1)
        fire_write(1, 1)
        fire_gather(3, 1)

        def body(i2, carry):
            for k in range(2):
                i = 2 * i2 + k
                drain_gather()
                drain_write()
                select_transpose(i, k)
                fire_write(i, k)
                fire_gather(i + 2, k)
            return carry

        lax.fori_loop(1, nblk // 2 - 1, body, 0)

        for i in (nblk - 2, nblk - 1):
            buf = i % 2
            drain_gather()
            drain_write()
            select_transpose(i, buf)
            fire_write(i, buf)
        drain_write()
        drain_write()

    return gather_kernel


def kernel(indices, W):
    batch, hist = indices.shape
    vocab, dim = W.shape
    idx_t = jnp.transpose(indices).astype(jnp.int32)
    w_pairs = jnp.reshape(W, (vocab * dim // (2 * DIM), 2 * DIM))
    out = _build(batch, hist, w_pairs.shape[0])(idx_t, w_pairs)
    return jnp.transpose(out, (2, 0, 1))
